# baseline (speedup 1.0000x reference)
import jax
import jax.numpy as jnp
from jax.experimental import pallas as pl


def _cdist(a, b):
    aa = jnp.sum(a * a, axis=-1)[:, :, None]
    bb = jnp.sum(b * b, axis=-1)[:, None, :]
    ab = jnp.einsum('bqd,bkd->bqk', a, b, precision=jax.lax.Precision.HIGHEST)
    return jnp.sqrt(jnp.maximum(aa + bb - 2.0 * ab, 1e-12))


def _conv(x, w, b):
    y = jax.lax.conv_general_dilated(x, w, (1, 1), 'SAME', dimension_numbers=('NCHW', 'OIHW', 'NCHW'), precision=jax.lax.Precision.HIGHEST)
    return y + b[None, :, None, None]


def kernel(seq, len_seq, conv1_w, conv1_b, conv2_w, conv2_b, conv3_w, conv3_b, lin1_w, lin1_b, lin2_w, lin2_b):
    B, T, D = seq.shape
    Tc = jnp.clip(len_seq, 8, T).astype(jnp.float32)
    t = jnp.arange(T, dtype=jnp.float32)
    mq = (t[None, :] < Tc[:, None]).astype(jnp.float32)
    R = ((t[None, :] + 1.0) / Tc[:, None]) * mq
    blur = 0.1 * seq[:, :-2, :] + 0.8 * seq[:, 1:-1, :] + 0.1 * seq[:, 2:, :]
    blur = jnp.pad(blur, ((0, 0), (0, 2), (0, 0)))
    outTc = Tc - 2.0
    mk = (t[None, :] < outTc[:, None]).astype(jnp.float32)
    avged_seq = blur * mk[:, :, None]
    avged_R = ((t[None, :] + 1.0) / outTc[:, None]) * mk
    Dm = _cdist(seq, avged_seq)
    Pm = jnp.abs(R[:, :, None] - avged_R[:, None, :])
    A = jnp.stack([Dm, Pm], axis=1)
    A = jax.nn.relu(_conv(A, conv1_w, conv1_b))
    A = jax.nn.relu(_conv(A, conv2_w, conv2_b))
    A = _conv(A, conv3_w, conv3_b)[:, 0] + Dm
    A = jax.nn.softmax(-A, axis=2)
    At = A.reshape(B, T * T)
    At = At / jnp.sum(jnp.abs(At), axis=1, keepdims=True)
    dis = jnp.einsum('bq,bq->b', Dm.reshape(B, T * T), At, precision=jax.lax.Precision.HIGHEST)
    h = jax.nn.relu(jax.lax.dot(seq.reshape(-1, D), lin1_w.T, precision=jax.lax.Precision.HIGHEST) + lin1_b)
    pred = jax.lax.dot(h, lin2_w.T, precision=jax.lax.Precision.HIGHEST) + lin2_b
    pred = pred / jnp.maximum(jnp.linalg.norm(pred, axis=-1, keepdims=True), 1e-12)
    pred = pred.reshape(B, T, -1)
    return (A, dis, pred, avged_seq)


# speedup vs baseline: 20.2121x; 20.2121x over previous
import functools

import jax
import jax.numpy as jnp
from jax.experimental import pallas as pl

B, T, D = 4, 512, 256
XP = 640
HT = 32
HALO = 8
RIN = HT + 2 * HALO
NT = T // HT


def _prep_body(seq_ref, len_ref, avg_ref, r_ref, avgr_ref, kk_ref):
    s = seq_ref[...]
    t = jax.lax.broadcasted_iota(jnp.int32, (B, T), 1).astype(jnp.float32)
    tc = jnp.clip(len_ref[...].astype(jnp.float32), 8.0, float(T))
    mq = (t < tc).astype(jnp.float32)
    r_ref[...] = ((t + 1.0) / tc) * mq
    out_tc = tc - 2.0
    mk = (t < out_tc).astype(jnp.float32)
    avgr_ref[...] = ((t + 1.0) / out_tc) * mk
    blur = 0.1 * s[:, :-2, :] + 0.8 * s[:, 1:-1, :] + 0.1 * s[:, 2:, :]
    blur = jnp.pad(blur, ((0, 0), (0, 2), (0, 0)))
    avg = blur * mk[:, :, None]
    avg_ref[...] = avg
    kk_ref[...] = jnp.sum(avg * avg, axis=2)


def _shift_x(v, s):
    if s == 0:
        return v
    if s > 0:
        return jnp.pad(v[:, :, s:], ((0, 0), (0, 0), (0, s)))
    return jnp.pad(v[:, :, :s], ((0, 0), (0, 0), (-s, 0)))


def _conv_mm(u, wp, bias, nco, taps, colmask):
    v = jax.lax.dot_general(wp, u, (((0,), (0,)), ((), ())),
                            preferred_element_type=jnp.float32)
    half = (taps - 1) // 2
    acc = _shift_x(v[0 * nco:1 * nco], 0 - half)
    for dx in range(1, taps):
        acc = acc + _shift_x(v[dx * nco:(dx + 1) * nco], dx - half)
    if bias is not None:
        acc = jnp.maximum(acc + bias, 0.0) * colmask
    return acc


def _main_body(seq_ref, avg_ref, r_ref, avgr_ref, kk_ref,
               w1_ref, b1_ref, w2_ref, b2_ref, w3_ref, b3_ref,
               a_ref, dis_ref):
    i = pl.program_id(1)
    sq = seq_ref[0, pl.ds(i * HT, RIN), :]
    rq = r_ref[0, pl.ds(i * HT, RIN), :]
    avg = avg_ref[0]
    avgr = avgr_ref[...][:, 0, :]
    kk = kk_ref[...][:, 0, :]

    qi = jax.lax.broadcasted_iota(jnp.int32, (RIN, 1), 0) + (i * HT - HALO)
    rmask = ((qi >= 0) & (qi < T)).astype(jnp.float32)

    qq = jnp.sum(sq * sq, axis=1, keepdims=True)
    mm = jax.lax.dot_general(sq.astype(jnp.bfloat16), avg.astype(jnp.bfloat16),
                             (((1,), (1,)), ((), ())),
                             preferred_element_type=jnp.float32)
    dq = jnp.sqrt(jnp.maximum(qq + kk - 2.0 * mm, 1e-12)) * rmask
    pq = jnp.abs(rq - avgr) * rmask

    d640 = jnp.pad(dq, ((0, 0), (2, XP - 514)))
    p640 = jnp.pad(pq, ((0, 0), (2, XP - 514)))
    dp = jnp.stack([d640, p640], axis=0)

    lane = jax.lax.broadcasted_iota(jnp.int32, (1, 1, XP), 2)
    colmask = ((lane >= 2) & (lane < 514)).astype(jnp.float32)

    b1 = b1_ref[...][:, :, None]
    b2 = b2_ref[...][:, :, None]

    u1 = jnp.concatenate([dp[:, dy:dy + RIN - 4, :] for dy in range(5)],
                         axis=0).astype(jnp.bfloat16)
    c1 = _conv_mm(u1, w1_ref[...], b1, 30, 5, colmask)

    u2 = jnp.concatenate([c1[:, dy:dy + RIN - 8, :] for dy in range(5)],
                         axis=0).astype(jnp.bfloat16)
    c2 = _conv_mm(u2, w2_ref[...], b2, 30, 5, colmask)

    u3 = jnp.concatenate([c2[:, dy:dy + RIN - 10, :] for dy in range(3)],
                         axis=0).astype(jnp.bfloat16)
    l3 = _conv_mm(u3, w3_ref[...], None, 1, 3, None)

    lc = l3[0, 3:3 + HT, 2:514] + b3_ref[0, 0] + dq[HALO:HALO + HT, :]
    neg = -lc
    mx = jnp.max(neg, axis=1, keepdims=True)
    e = jnp.exp(neg - mx)
    a = e / jnp.sum(e, axis=1, keepdims=True)
    a_ref[0] = a

    p = jnp.sum(dq[HALO:HALO + HT, :] * a, keepdims=True) * (1.0 / float(T))

    @pl.when(i == 0)
    def _():
        dis_ref[0] = p

    @pl.when(i != 0)
    def _():
        dis_ref[0] += p


def _head_body(seq_ref, w1_ref, b1_ref, w2_ref, b2_ref, out_ref):
    s = seq_ref[...].reshape(B * T, D).astype(jnp.bfloat16)
    h = jax.lax.dot_general(s, w1_ref[...], (((1,), (0,)), ((), ())),
                            preferred_element_type=jnp.float32)
    h = jnp.maximum(h + b1_ref[...], 0.0).astype(jnp.bfloat16)
    p = jax.lax.dot_general(h, w2_ref[...], (((1,), (0,)), ((), ())),
                            preferred_element_type=jnp.float32)
    p = p + b2_ref[...]
    nrm = jnp.maximum(jnp.sqrt(jnp.sum(p * p, axis=1, keepdims=True)), 1e-12)
    out_ref[...] = (p / nrm).reshape(B, T, D)


@functools.partial(jax.jit, static_argnames=())
def kernel(seq, len_seq, conv1_w, conv1_b, conv2_w, conv2_b, conv3_w, conv3_b,
           lin1_w, lin1_b, lin2_w, lin2_b):
    f32 = jnp.float32
    avg, r, avgr, kk = pl.pallas_call(
        _prep_body,
        out_shape=(
            jax.ShapeDtypeStruct((B, T, D), f32),
            jax.ShapeDtypeStruct((B, T), f32),
            jax.ShapeDtypeStruct((B, T), f32),
            jax.ShapeDtypeStruct((B, T), f32),
        ),
    )(seq, len_seq.reshape(B, 1))

    seq_pad = jnp.pad(seq, ((0, 0), (HALO, HALO), (0, 0)))
    r_pad = jnp.pad(r, ((0, 0), (HALO, HALO)))[:, :, None]

    w1p = conv1_w.transpose(2, 1, 3, 0).reshape(10, 150).astype(jnp.bfloat16)
    w2p = conv2_w.transpose(2, 1, 3, 0).reshape(150, 150).astype(jnp.bfloat16)
    w3p = conv3_w.transpose(2, 1, 3, 0).reshape(90, 3).astype(jnp.bfloat16)

    full = lambda shape: pl.BlockSpec(shape, lambda b, i: (0,) * len(shape))
    a_out, dis_part = pl.pallas_call(
        _main_body,
        grid=(B, NT),
        in_specs=[
            pl.BlockSpec((1, T + 2 * HALO, D), lambda b, i: (b, 0, 0)),
            pl.BlockSpec((1, T, D), lambda b, i: (b, 0, 0)),
            pl.BlockSpec((1, T + 2 * HALO, 1), lambda b, i: (b, 0, 0)),
            pl.BlockSpec((1, 1, T), lambda b, i: (b, 0, 0)),
            pl.BlockSpec((1, 1, T), lambda b, i: (b, 0, 0)),
            full((10, 150)),
            full((30, 1)),
            full((150, 150)),
            full((30, 1)),
            full((90, 3)),
            full((1, 1)),
        ],
        out_specs=(
            pl.BlockSpec((1, HT, T), lambda b, i: (b, i, 0)),
            pl.BlockSpec((1, 1, 1), lambda b, i: (b, 0, 0)),
        ),
        out_shape=(
            jax.ShapeDtypeStruct((B, T, T), f32),
            jax.ShapeDtypeStruct((B, 1, 1), f32),
        ),
    )(seq_pad, avg, r_pad, avgr[:, None, :], kk[:, None, :],
      w1p, conv1_b.reshape(30, 1), w2p, conv2_b.reshape(30, 1), w3p,
      conv3_b.reshape(1, 1))

    pred = pl.pallas_call(
        _head_body,
        out_shape=jax.ShapeDtypeStruct((B, T, D), f32),
    )(seq, lin1_w.T.astype(jnp.bfloat16), lin1_b.reshape(1, D),
      lin2_w.T.astype(jnp.bfloat16), lin2_b.reshape(1, D))

    return (a_out, dis_part[:, 0, 0], pred, avg)


# speedup vs baseline: 38.9951x; 1.9293x over previous
import jax
import jax.numpy as jnp
from jax.experimental import pallas as pl
from jax.experimental.pallas import tpu as pltpu

B, T, D = 4, 512, 256
XP = 640
HT = 32
HALO = 8
RIN = HT + 2 * HALO
NT = T // HT
M0 = RIN * XP
M1 = (RIN - 4) * XP
M2 = (RIN - 8) * XP
M3 = (RIN - 10) * XP


def _prep_body(seq_ref, len_ref, avg_ref, r_ref, avgr_ref, kk_ref):
    s = seq_ref[...]
    t = jax.lax.broadcasted_iota(jnp.int32, (B, T), 1).astype(jnp.float32)
    tc = jnp.clip(len_ref[...].astype(jnp.float32), 8.0, float(T))
    mq = (t < tc).astype(jnp.float32)
    r_ref[...] = ((t + 1.0) / tc) * mq
    out_tc = tc - 2.0
    mk = (t < out_tc).astype(jnp.float32)
    avgr_ref[...] = ((t + 1.0) / out_tc) * mk
    blur = 0.1 * s[:, :-2, :] + 0.8 * s[:, 1:-1, :] + 0.1 * s[:, 2:, :]
    blur = jnp.pad(blur, ((0, 0), (0, 2), (0, 0)))
    avg = blur * mk[:, :, None]
    avg_ref[...] = avg
    kk_ref[...] = jnp.sum(avg * avg, axis=2)


def _conv_flat(u, wp, bias, taps, stride, colmask):
    v = jax.lax.dot_general(wp, u, (((0,), (0,)), ((), ())),
                            preferred_element_type=jnp.float32)
    m = u.shape[1]
    half = (taps - 1) // 2
    acc = pltpu.roll(v[0:stride], (-(0 - half)) % m, 1)
    for dx in range(1, taps):
        s = dx - half
        part = v[dx * stride:(dx + 1) * stride]
        acc = acc + (part if s == 0 else pltpu.roll(part, (-s) % m, 1))
    if bias is not None:
        acc = jnp.maximum(acc + bias, 0.0) * colmask
    return acc


def _main_body(seq_ref, avg_ref, r_ref, avgr_ref, kk_ref,
               w1_ref, b1_ref, w2_ref, b2_ref, w3_ref, b3_ref,
               a_ref, dis_ref):
    i = pl.program_id(1)
    sq = seq_ref[0, pl.ds(i * HT, RIN), :]
    rq = r_ref[0, pl.ds(i * HT, RIN), :]
    avg = avg_ref[0]
    avgr = avgr_ref[...][:, 0, :]
    kk = kk_ref[...][:, 0, :]

    qi = jax.lax.broadcasted_iota(jnp.int32, (RIN, 1), 0) + (i * HT - HALO)
    rmask = ((qi >= 0) & (qi < T)).astype(jnp.float32)

    qq = jnp.sum(sq * sq, axis=1, keepdims=True)
    mm = jax.lax.dot_general(sq.astype(jnp.bfloat16), avg.astype(jnp.bfloat16),
                             (((1,), (1,)), ((), ())),
                             preferred_element_type=jnp.float32)
    dq = jnp.sqrt(jnp.maximum(qq + kk - 2.0 * mm, 1e-12)) * rmask
    pq = jnp.abs(rq - avgr) * rmask

    df = jnp.pad(dq, ((0, 0), (0, XP - T))).reshape(1, M0)
    pf = jnp.pad(pq, ((0, 0), (0, XP - T))).reshape(1, M0)
    dp8 = jnp.concatenate([df, pf, jnp.zeros((6, M0), jnp.float32)],
                          axis=0).astype(jnp.bfloat16)

    lane = jax.lax.broadcasted_iota(jnp.int32, (1, M1), 1).astype(jnp.float32)
    xcol = lane - float(XP) * jnp.floor(lane * (1.0 / XP))
    colmask = (xcol < float(T)).astype(jnp.float32)

    b1 = b1_ref[...]
    b2 = b2_ref[...]

    u1 = jnp.concatenate([dp8[:, dy * XP:dy * XP + M1] for dy in range(5)],
                         axis=0)
    c1 = _conv_flat(u1, w1_ref[...], b1, 5, 32, colmask)

    c1b = c1.astype(jnp.bfloat16)
    u2 = jnp.concatenate([c1b[:, dy * XP:dy * XP + M2] for dy in range(5)],
                         axis=0)
    c2 = _conv_flat(u2, w2_ref[...], b2, 5, 32, colmask[:, :M2])

    c2b = c2.astype(jnp.bfloat16)
    u3 = jnp.concatenate([c2b[:, dy * XP:dy * XP + M3] for dy in range(3)],
                         axis=0)
    v3 = _conv_flat(u3, w3_ref[...], None, 3, 8, None)

    lc = v3[0:1, 3 * XP:3 * XP + HT * XP].reshape(HT, XP)[:, :T]
    lc = lc + b3_ref[0, 0] + dq[HALO:HALO + HT, :]
    neg = -lc
    mx = jnp.max(neg, axis=1, keepdims=True)
    e = jnp.exp(neg - mx)
    a = e / jnp.sum(e, axis=1, keepdims=True)
    a_ref[0] = a

    p = jnp.sum(dq[HALO:HALO + HT, :] * a, keepdims=True) * (1.0 / float(T))

    @pl.when(i == 0)
    def _():
        dis_ref[0] = p

    @pl.when(i != 0)
    def _():
        dis_ref[0] += p


def _head_body(seq_ref, w1_ref, b1_ref, w2_ref, b2_ref, out_ref):
    s = seq_ref[...].reshape(B * T, D).astype(jnp.bfloat16)
    h = jax.lax.dot_general(s, w1_ref[...], (((1,), (0,)), ((), ())),
                            preferred_element_type=jnp.float32)
    h = jnp.maximum(h + b1_ref[...], 0.0).astype(jnp.bfloat16)
    p = jax.lax.dot_general(h, w2_ref[...], (((1,), (0,)), ((), ())),
                            preferred_element_type=jnp.float32)
    p = p + b2_ref[...]
    nrm = jnp.maximum(jnp.sqrt(jnp.sum(p * p, axis=1, keepdims=True)), 1e-12)
    out_ref[...] = (p / nrm).reshape(B, T, D)


def kernel(seq, len_seq, conv1_w, conv1_b, conv2_w, conv2_b, conv3_w, conv3_b,
           lin1_w, lin1_b, lin2_w, lin2_b):
    f32 = jnp.float32
    bf16 = jnp.bfloat16
    avg, r, avgr, kk = pl.pallas_call(
        _prep_body,
        out_shape=(
            jax.ShapeDtypeStruct((B, T, D), f32),
            jax.ShapeDtypeStruct((B, T), f32),
            jax.ShapeDtypeStruct((B, T), f32),
            jax.ShapeDtypeStruct((B, T), f32),
        ),
    )(seq, len_seq.reshape(B, 1))

    seq_pad = jnp.pad(seq, ((0, 0), (HALO, HALO), (0, 0)))
    r_pad = jnp.pad(r, ((0, 0), (HALO, HALO)))[:, :, None]

    w1p = jnp.zeros((5, 8, 5, 32), f32).at[:, :2, :, :30].set(
        conv1_w.transpose(2, 1, 3, 0)).reshape(40, 160).astype(bf16)
    w2p = jnp.zeros((5, 32, 5, 32), f32).at[:, :30, :, :30].set(
        conv2_w.transpose(2, 1, 3, 0)).reshape(160, 160).astype(bf16)
    w3p = jnp.zeros((3, 32, 3, 8), f32).at[:, :30, :, :1].set(
        conv3_w.transpose(2, 1, 3, 0)).reshape(96, 24).astype(bf16)
    b1p = jnp.pad(conv1_b.reshape(30, 1), ((0, 2), (0, 0)))
    b2p = jnp.pad(conv2_b.reshape(30, 1), ((0, 2), (0, 0)))

    full = lambda shape: pl.BlockSpec(shape, lambda b, i: (0,) * len(shape))
    a_out, dis_part = pl.pallas_call(
        _main_body,
        grid=(B, NT),
        in_specs=[
            pl.BlockSpec((1, T + 2 * HALO, D), lambda b, i: (b, 0, 0)),
            pl.BlockSpec((1, T, D), lambda b, i: (b, 0, 0)),
            pl.BlockSpec((1, T + 2 * HALO, 1), lambda b, i: (b, 0, 0)),
            pl.BlockSpec((1, 1, T), lambda b, i: (b, 0, 0)),
            pl.BlockSpec((1, 1, T), lambda b, i: (b, 0, 0)),
            full((40, 160)),
            full((32, 1)),
            full((160, 160)),
            full((32, 1)),
            full((96, 24)),
            full((1, 1)),
        ],
        out_specs=(
            pl.BlockSpec((1, HT, T), lambda b, i: (b, i, 0)),
            pl.BlockSpec((1, 1, 1), lambda b, i: (b, 0, 0)),
        ),
        out_shape=(
            jax.ShapeDtypeStruct((B, T, T), f32),
            jax.ShapeDtypeStruct((B, 1, 1), f32),
        ),
    )(seq_pad, avg, r_pad, avgr[:, None, :], kk[:, None, :],
      w1p, b1p, w2p, b2p, w3p, conv3_b.reshape(1, 1))

    pred = pl.pallas_call(
        _head_body,
        out_shape=jax.ShapeDtypeStruct((B, T, D), f32),
    )(seq, lin1_w.T.astype(bf16), lin1_b.reshape(1, D),
      lin2_w.T.astype(bf16), lin2_b.reshape(1, D))

    return (a_out, dis_part[:, 0, 0], pred, avg)


# speedup vs baseline: 42.4573x; 1.0888x over previous
import jax
import jax.numpy as jnp
from jax.experimental import pallas as pl
from jax.experimental.pallas import tpu as pltpu

B, T, D = 4, 512, 256
XP = 640
HT = 64
HALO = 8
RIN = HT + 2 * HALO
NT = T // HT
M0 = RIN * XP
M1 = (RIN - 4) * XP
M2 = (RIN - 8) * XP
M3 = (RIN - 10) * XP


def _prep_body(seq_ref, len_ref, avg_ref, r_ref, avgr_ref, kk_ref):
    s = seq_ref[...]
    t = jax.lax.broadcasted_iota(jnp.int32, (B, T), 1).astype(jnp.float32)
    tc = jnp.clip(len_ref[...].astype(jnp.float32), 8.0, float(T))
    mq = (t < tc).astype(jnp.float32)
    r_ref[...] = ((t + 1.0) / tc) * mq
    out_tc = tc - 2.0
    mk = (t < out_tc).astype(jnp.float32)
    avgr_ref[...] = ((t + 1.0) / out_tc) * mk
    blur = 0.1 * s[:, :-2, :] + 0.8 * s[:, 1:-1, :] + 0.1 * s[:, 2:, :]
    blur = jnp.pad(blur, ((0, 0), (0, 2), (0, 0)))
    avg = blur * mk[:, :, None]
    avg_ref[...] = avg
    kk_ref[...] = jnp.sum(avg * avg, axis=2)


def _conv_flat(u, wp, bias, taps, stride, colmask):
    v = jax.lax.dot_general(wp, u, (((0,), (0,)), ((), ())),
                            preferred_element_type=jnp.float32)
    m = u.shape[1]
    half = (taps - 1) // 2
    acc = pltpu.roll(v[0:stride], (-(0 - half)) % m, 1)
    for dx in range(1, taps):
        s = dx - half
        part = v[dx * stride:(dx + 1) * stride]
        acc = acc + (part if s == 0 else pltpu.roll(part, (-s) % m, 1))
    if bias is not None:
        acc = jnp.maximum(acc + bias, 0.0) * colmask
    return acc


def _main_body(seq_ref, avg_ref, r_ref, avgr_ref, kk_ref,
               w1_ref, b1_ref, w2_ref, b2_ref, w3_ref, b3_ref,
               a_ref, dis_ref):
    i = pl.program_id(1)
    sq = seq_ref[0, pl.ds(i * HT, RIN), :]
    rq = r_ref[0, pl.ds(i * HT, RIN), :]
    avg = avg_ref[0]
    avgr = avgr_ref[...][:, 0, :]
    kk = kk_ref[...][:, 0, :]

    qi = jax.lax.broadcasted_iota(jnp.int32, (RIN, 1), 0) + (i * HT - HALO)
    rmask = ((qi >= 0) & (qi < T)).astype(jnp.float32)

    qq = jnp.sum(sq * sq, axis=1, keepdims=True)
    mm = jax.lax.dot_general(sq.astype(jnp.bfloat16), avg.astype(jnp.bfloat16),
                             (((1,), (1,)), ((), ())),
                             preferred_element_type=jnp.float32)
    dq = jnp.sqrt(jnp.maximum(qq + kk - 2.0 * mm, 1e-12)) * rmask
    pq = jnp.abs(rq - avgr) * rmask

    df = jnp.pad(dq, ((0, 0), (0, XP - T))).reshape(1, M0)
    pf = jnp.pad(pq, ((0, 0), (0, XP - T))).reshape(1, M0)
    dp8 = jnp.concatenate([df, pf, jnp.zeros((6, M0), jnp.float32)],
                          axis=0).astype(jnp.bfloat16)

    lane = jax.lax.broadcasted_iota(jnp.int32, (1, M1), 1).astype(jnp.float32)
    xcol = lane - float(XP) * jnp.floor(lane * (1.0 / XP))
    colmask = (xcol < float(T)).astype(jnp.float32)

    b1 = b1_ref[...]
    b2 = b2_ref[...]

    u1 = jnp.concatenate([dp8[:, dy * XP:dy * XP + M1] for dy in range(5)],
                         axis=0)
    c1 = _conv_flat(u1, w1_ref[...], b1, 5, 32, colmask)

    c1b = c1.astype(jnp.bfloat16)
    u2 = jnp.concatenate([c1b[:, dy * XP:dy * XP + M2] for dy in range(5)],
                         axis=0)
    c2 = _conv_flat(u2, w2_ref[...], b2, 5, 32, colmask[:, :M2])

    c2b = c2.astype(jnp.bfloat16)
    u3 = jnp.concatenate([c2b[:, dy * XP:dy * XP + M3] for dy in range(3)],
                         axis=0)
    v3 = _conv_flat(u3, w3_ref[...], None, 3, 8, None)

    lc = v3[0:1, 3 * XP:3 * XP + HT * XP].reshape(HT, XP)[:, :T]
    lc = lc + b3_ref[0, 0] + dq[HALO:HALO + HT, :]
    neg = -lc
    mx = jnp.max(neg, axis=1, keepdims=True)
    e = jnp.exp(neg - mx)
    a = e / jnp.sum(e, axis=1, keepdims=True)
    a_ref[0] = a

    p = jnp.sum(dq[HALO:HALO + HT, :] * a, keepdims=True) * (1.0 / float(T))

    @pl.when(i == 0)
    def _():
        dis_ref[0] = p

    @pl.when(i != 0)
    def _():
        dis_ref[0] += p


def _head_body(seq_ref, w1_ref, b1_ref, w2_ref, b2_ref, out_ref):
    s = seq_ref[...].reshape(B * T, D).astype(jnp.bfloat16)
    h = jax.lax.dot_general(s, w1_ref[...], (((1,), (0,)), ((), ())),
                            preferred_element_type=jnp.float32)
    h = jnp.maximum(h + b1_ref[...], 0.0).astype(jnp.bfloat16)
    p = jax.lax.dot_general(h, w2_ref[...], (((1,), (0,)), ((), ())),
                            preferred_element_type=jnp.float32)
    p = p + b2_ref[...]
    nrm = jnp.maximum(jnp.sqrt(jnp.sum(p * p, axis=1, keepdims=True)), 1e-12)
    out_ref[...] = (p / nrm).reshape(B, T, D)


def kernel(seq, len_seq, conv1_w, conv1_b, conv2_w, conv2_b, conv3_w, conv3_b,
           lin1_w, lin1_b, lin2_w, lin2_b):
    f32 = jnp.float32
    bf16 = jnp.bfloat16
    avg, r, avgr, kk = pl.pallas_call(
        _prep_body,
        out_shape=(
            jax.ShapeDtypeStruct((B, T, D), f32),
            jax.ShapeDtypeStruct((B, T), f32),
            jax.ShapeDtypeStruct((B, T), f32),
            jax.ShapeDtypeStruct((B, T), f32),
        ),
    )(seq, len_seq.reshape(B, 1))

    seq_pad = jnp.pad(seq, ((0, 0), (HALO, HALO), (0, 0)))
    r_pad = jnp.pad(r, ((0, 0), (HALO, HALO)))[:, :, None]

    w1p = jnp.zeros((5, 8, 5, 32), f32).at[:, :2, :, :30].set(
        conv1_w.transpose(2, 1, 3, 0)).reshape(40, 160).astype(bf16)
    w2p = jnp.zeros((5, 32, 5, 32), f32).at[:, :30, :, :30].set(
        conv2_w.transpose(2, 1, 3, 0)).reshape(160, 160).astype(bf16)
    w3p = jnp.zeros((3, 32, 3, 8), f32).at[:, :30, :, :1].set(
        conv3_w.transpose(2, 1, 3, 0)).reshape(96, 24).astype(bf16)
    b1p = jnp.pad(conv1_b.reshape(30, 1), ((0, 2), (0, 0)))
    b2p = jnp.pad(conv2_b.reshape(30, 1), ((0, 2), (0, 0)))

    full = lambda shape: pl.BlockSpec(shape, lambda b, i: (0,) * len(shape))
    a_out, dis_part = pl.pallas_call(
        _main_body,
        grid=(B, NT),
        in_specs=[
            pl.BlockSpec((1, T + 2 * HALO, D), lambda b, i: (b, 0, 0)),
            pl.BlockSpec((1, T, D), lambda b, i: (b, 0, 0)),
            pl.BlockSpec((1, T + 2 * HALO, 1), lambda b, i: (b, 0, 0)),
            pl.BlockSpec((1, 1, T), lambda b, i: (b, 0, 0)),
            pl.BlockSpec((1, 1, T), lambda b, i: (b, 0, 0)),
            full((40, 160)),
            full((32, 1)),
            full((160, 160)),
            full((32, 1)),
            full((96, 24)),
            full((1, 1)),
        ],
        out_specs=(
            pl.BlockSpec((1, HT, T), lambda b, i: (b, i, 0)),
            pl.BlockSpec((1, 1, 1), lambda b, i: (b, 0, 0)),
        ),
        out_shape=(
            jax.ShapeDtypeStruct((B, T, T), f32),
            jax.ShapeDtypeStruct((B, 1, 1), f32),
        ),
    )(seq_pad, avg, r_pad, avgr[:, None, :], kk[:, None, :],
      w1p, b1p, w2p, b2p, w3p, conv3_b.reshape(1, 1))

    pred = pl.pallas_call(
        _head_body,
        out_shape=jax.ShapeDtypeStruct((B, T, D), f32),
    )(seq, lin1_w.T.astype(bf16), lin1_b.reshape(1, D),
      lin2_w.T.astype(bf16), lin2_b.reshape(1, D))

    return (a_out, dis_part[:, 0, 0], pred, avg)
